# baseline (speedup 1.0000x reference)
import jax
import jax.numpy as jnp
from jax.experimental import pallas as pl
from jax.experimental.pallas import tpu as pltpu

NSPLIT = 1


def _sage_kern(*refs):
    a_refs = refs[:NSPLIT]
    x_ref, xr_ref, w_ref, o_ref = refs[NSPLIT:]
    xb = x_ref[0]
    w = w_ref[...]
    tm = a_refs[0].shape[1]
    for k, a_ref in enumerate(a_refs):
        a = a_ref[0]
        tmp = jnp.dot(a, xb, preferred_element_type=jnp.float32)
        res = tmp + xr_ref[0, k * tm:(k + 1) * tm]
        o_ref[0, k * tm:(k + 1) * tm] = jnp.dot(
            res, w, preferred_element_type=jnp.float32)


def kernel(x, adj_t, W):
    B, S, _ = adj_t.shape
    N, IN = x.shape
    OUT = W.shape[1]
    TM = 2048
    TMS = TM // NSPLIT
    xb = x.reshape(B, S, IN)

    def a_spec(k):
        return pl.BlockSpec((1, TMS, S), lambda b, i, k=k: (b, NSPLIT * i + k, 0))

    out = pl.pallas_call(
        _sage_kern,
        grid=(B, S // TM),
        in_specs=[a_spec(k) for k in range(NSPLIT)] + [
            pl.BlockSpec((1, S, IN), lambda b, i: (b, 0, 0)),
            pl.BlockSpec((1, TM, IN), lambda b, i: (b, i, 0)),
            pl.BlockSpec((IN, OUT), lambda b, i: (0, 0)),
        ],
        out_specs=pl.BlockSpec((1, TM, OUT), lambda b, i: (b, i, 0)),
        out_shape=jax.ShapeDtypeStruct((B, S, OUT), jnp.float32),
        compiler_params=pltpu.CompilerParams(
            dimension_semantics=("parallel", "parallel"),
        ),
    )(*([adj_t] * NSPLIT), xb, xb, W)
    return out.reshape(N, OUT)


# speedup vs baseline: 1.0522x; 1.0522x over previous
import jax
import jax.numpy as jnp
from jax import lax
from jax.experimental import pallas as pl
from jax.experimental.pallas import tpu as pltpu

TM = 512
NBUF = 4


def _sage_kern(adj_hbm, x_ref, w_ref, o_ref, buf, sem):
    n_rows, S = adj_hbm.shape
    num_chunks = n_rows // TM
    blocks_per_batch = S // TM
    w = w_ref[...]

    def chunk_copy(i, slot):
        return pltpu.make_async_copy(
            adj_hbm.at[pl.ds(i * TM, TM), :],
            buf.at[slot],
            sem.at[slot],
        )

    for k in range(NBUF - 1):
        chunk_copy(k, k).start()

    def body(i, _):
        slot = lax.rem(i, NBUF)
        chunk_copy(i, slot).wait()
        nxt = i + NBUF - 1
        @pl.when(nxt < num_chunks)
        def _start_next():
            chunk_copy(nxt, lax.rem(nxt, NBUF)).start()
        b = lax.div(i, blocks_per_batch)
        xb = x_ref[pl.ds(b * S, S), :]
        a = buf[slot]
        tmp = jnp.dot(a, xb, preferred_element_type=jnp.float32)
        res = tmp + x_ref[pl.ds(i * TM, TM), :]
        o_ref[pl.ds(i * TM, TM), :] = jnp.dot(
            res, w, preferred_element_type=jnp.float32)
        return 0

    lax.fori_loop(0, num_chunks, body, 0)


def kernel(x, adj_t, W):
    B, S, _ = adj_t.shape
    N, IN = x.shape
    OUT = W.shape[1]
    adj2d = adj_t.reshape(N, S)

    out = pl.pallas_call(
        _sage_kern,
        in_specs=[
            pl.BlockSpec(memory_space=pltpu.MemorySpace.HBM),
            pl.BlockSpec(memory_space=pltpu.MemorySpace.VMEM),
            pl.BlockSpec(memory_space=pltpu.MemorySpace.VMEM),
        ],
        out_specs=pl.BlockSpec(memory_space=pltpu.MemorySpace.VMEM),
        out_shape=jax.ShapeDtypeStruct((N, OUT), jnp.float32),
        scratch_shapes=[
            pltpu.VMEM((NBUF, TM, S), jnp.float32),
            pltpu.SemaphoreType.DMA((NBUF,)),
        ],
    )(adj2d, x, W)
    return out
